# baseline (speedup 1.0000x reference)
import jax
import jax.numpy as jnp
from jax import lax
from jax.experimental import pallas as pl
from jax.experimental.pallas import tpu as pltpu

_TOPK = 80
_RADIUS_GAIN = 14.0
_RADIUS_MIN = 1.5
_RADIUS_MAX = 18.0


def _shift(x, d, axis, fill):
    h, w = x.shape
    if d == 0:
        return x
    if axis == 1:
        pad = jnp.full((h, abs(d)), fill, x.dtype)
        if d > 0:
            return jnp.concatenate([pad, x[:, : w - d]], axis=1)
        return jnp.concatenate([x[:, -d:], pad], axis=1)
    pad = jnp.full((abs(d), w), fill, x.dtype)
    if d > 0:
        return jnp.concatenate([pad, x[: h - d, :]], axis=0)
    return jnp.concatenate([x[-d:, :], pad], axis=0)


def _sum9(x, axis):
    acc = x
    for d in (-4, -3, -2, -1, 1, 2, 3, 4):
        acc = acc + _shift(x, d, axis, 0.0)
    return acc


def _max5(x, axis):
    ninf = -jnp.inf
    m3 = jnp.maximum(x, jnp.maximum(_shift(x, 1, axis, ninf),
                                    _shift(x, -1, axis, ninf)))
    return jnp.maximum(_shift(m3, 1, axis, ninf), _shift(m3, -1, axis, ninf))


def _body(vs_ref, dp_ref, out_ref, mx_ref, rm_ref, b_ref,
          lv_s, rf_s, ic_s, tlo_s, thi_s):
    h, w = mx_ref.shape
    p = vs_ref[0]

    occ = (p > 0.2).astype(jnp.float32)
    so = _sum9(_sum9(occ, 1), 0)
    sm = _sum9(_sum9(p, 1), 0)
    prod = so * sm * (1.0 / (81.0 * 81.0))
    mn = jnp.min(prod)
    mxv = jnp.max(prod)
    compact = (prod - mn) / (mxv - mn + 1e-6)
    score = p * compact

    pooled = _max5(_max5(score, 1), 0)
    maxima = jnp.where(score == pooled, score, 0.0)
    mx_ref[:, :] = maxima
    rm_ref[:, :] = jnp.max(maxima, axis=1, keepdims=True)

    riota = lax.broadcasted_iota(jnp.int32, (h, 1), 0)
    ciota = lax.broadcasted_iota(jnp.int32, (1, w), 1)
    xxr = ciota.astype(jnp.float32)

    out_ref[0] = jnp.full((h, w), -jnp.inf, jnp.float32)

    def step(i, carry):
        rm = rm_ref[:, :]
        m = jnp.max(rm)
        r = jnp.min(jnp.where(rm == m, riota, h))
        row = mx_ref[pl.ds(r, 1), :]
        c = jnp.min(jnp.where(row == m, ciota, w))
        newrow = jnp.where(ciota == c, 0.0, row)
        mx_ref[pl.ds(r, 1), :] = newrow
        rm_ref[pl.ds(r, 1), :] = jnp.max(newrow, axis=1, keepdims=True)
        drow = dp_ref[0, pl.ds(r, 1), :]
        z = jnp.sum(jnp.where(ciota == c, drow, 0.0))
        z = jnp.maximum(z, 0.001)
        radius = jnp.clip(_RADIUS_GAIN / z, _RADIUS_MIN, _RADIUS_MAX)
        sig2 = (0.6 * radius) ** 2
        invc = 1.0 / (2.0 * sig2 + 1e-6)
        lv_s[i] = jnp.log(m)
        rf_s[i] = r.astype(jnp.float32)
        ic_s[i] = invc
        b_ref[pl.ds(i, 1), :] = (xxr - c.astype(jnp.float32)) ** 2 * invc
        d = jnp.ceil(3.3 * radius).astype(jnp.int32)
        tlo = jnp.maximum(r - d, 0) // 8
        thi = (jnp.minimum(r + d + 1, h) + 7) // 8
        pos = m > 0.0
        tlo_s[i] = jnp.where(pos, tlo, 0)
        thi_s[i] = jnp.where(pos, thi, 0)
        return carry

    lax.fori_loop(0, _TOPK, step, 0, unroll=False)

    yy8 = lax.broadcasted_iota(jnp.int32, (8, 1), 0).astype(jnp.float32)

    def kstep(k, carry):
        lv = lv_s[k]
        rf = rf_s[k]
        ic = ic_s[k]
        brow = b_ref[pl.ds(k, 1), :]

        def tstep(t, c2):
            base = t * 8
            dy = yy8 + base.astype(jnp.float32) - rf
            ay = lv - dy * dy * ic
            tile = out_ref[0, pl.ds(base, 8), :]
            out_ref[0, pl.ds(base, 8), :] = jnp.maximum(tile, ay - brow)
            return c2

        lax.fori_loop(tlo_s[k], thi_s[k], tstep, 0)
        return carry

    lax.fori_loop(0, _TOPK, kstep, 0, unroll=False)

    g = jnp.exp(out_ref[0])
    gmn = jnp.min(g)
    gmx = jnp.max(g)
    out_ref[0] = (g - gmn) / (gmx - gmn + 1e-6)


def kernel(voxel_score_map, depth):
    b, ch, h, w = voxel_score_map.shape
    vs = voxel_score_map.reshape(b, h, w)
    dp = depth.reshape(b, h, w)
    out = pl.pallas_call(
        _body,
        grid=(b,),
        in_specs=[
            pl.BlockSpec((1, h, w), lambda i: (i, 0, 0)),
            pl.BlockSpec((1, h, w), lambda i: (i, 0, 0)),
        ],
        out_specs=pl.BlockSpec((1, h, w), lambda i: (i, 0, 0)),
        out_shape=jax.ShapeDtypeStruct((b, h, w), jnp.float32),
        scratch_shapes=[
            pltpu.VMEM((h, w), jnp.float32),
            pltpu.VMEM((h, 1), jnp.float32),
            pltpu.VMEM((_TOPK, w), jnp.float32),
            pltpu.SMEM((_TOPK,), jnp.float32),
            pltpu.SMEM((_TOPK,), jnp.float32),
            pltpu.SMEM((_TOPK,), jnp.float32),
            pltpu.SMEM((_TOPK,), jnp.int32),
            pltpu.SMEM((_TOPK,), jnp.int32),
        ],
        compiler_params=pltpu.CompilerParams(
            dimension_semantics=("arbitrary",),
        ),
    )(vs, dp)
    return out.reshape(b, ch, h, w)


# speedup vs baseline: 1.2694x; 1.2694x over previous
import jax
import jax.numpy as jnp
from jax import lax
from jax.experimental import pallas as pl
from jax.experimental.pallas import tpu as pltpu

_TOPK = 80
_RADIUS_GAIN = 14.0
_RADIUS_MIN = 1.5
_RADIUS_MAX = 18.0


def _shift(x, d, axis, fill):
    h, w = x.shape
    if d == 0:
        return x
    if axis == 1:
        pad = jnp.full((h, abs(d)), fill, x.dtype)
        if d > 0:
            return jnp.concatenate([pad, x[:, : w - d]], axis=1)
        return jnp.concatenate([x[:, -d:], pad], axis=1)
    pad = jnp.full((abs(d), w), fill, x.dtype)
    if d > 0:
        return jnp.concatenate([pad, x[: h - d, :]], axis=0)
    return jnp.concatenate([x[-d:, :], pad], axis=0)


def _sum9(x, axis):
    acc = x
    for d in (-4, -3, -2, -1, 1, 2, 3, 4):
        acc = acc + _shift(x, d, axis, 0.0)
    return acc


def _max5(x, axis):
    ninf = -jnp.inf
    m3 = jnp.maximum(x, jnp.maximum(_shift(x, 1, axis, ninf),
                                    _shift(x, -1, axis, ninf)))
    return jnp.maximum(_shift(m3, 1, axis, ninf), _shift(m3, -1, axis, ninf))


def _body(vs_ref, dp_ref, out_ref, mx_ref):
    h, w = mx_ref.shape
    p = vs_ref[0]

    occ = (p > 0.2).astype(jnp.float32)
    so = _sum9(_sum9(occ, 1), 0)
    sm = _sum9(_sum9(p, 1), 0)
    prod = so * sm * (1.0 / (81.0 * 81.0))
    mn = jnp.min(prod)
    mxv = jnp.max(prod)
    compact = (prod - mn) / (mxv - mn + 1e-6)
    score = p * compact

    pooled = _max5(_max5(score, 1), 0)
    maxima = jnp.where(score == pooled, score, 0.0)
    mx_ref[:, :] = maxima
    nb = h // 8
    rm0 = jnp.max(maxima.reshape(8, nb, w), axis=2)

    ciota = lax.broadcasted_iota(jnp.int32, (1, w), 1)
    xxr = ciota.astype(jnp.float32)
    fiota = (lax.broadcasted_iota(jnp.int32, (8, nb), 0) * nb
             + lax.broadcasted_iota(jnp.int32, (8, nb), 1))
    slab = 136
    yy_sl = lax.broadcasted_iota(jnp.int32, (slab, 1), 0)

    out_ref[0] = jnp.full((h, w), -jnp.inf, jnp.float32)

    def splat(lv, rf, ic, cf, base):
        base = pl.multiple_of(base, 8)
        ay = lv - ((yy_sl + base).astype(jnp.float32) - rf) ** 2 * ic
        bx = (xxr - cf) ** 2 * ic
        cur = out_ref[0, pl.ds(base, slab), :]
        out_ref[0, pl.ds(base, slab), :] = jnp.maximum(cur, ay - bx)

    def step(i, carry):
        rm, plv, prf, pic, pcf, pbase = carry
        m = jnp.max(rm)
        r = jnp.min(jnp.where(rm == m, fiota, h))
        row = mx_ref[pl.ds(r, 1), :]
        c = jnp.min(jnp.where(row == m, ciota, w))
        newrow = jnp.where(ciota == c, 0.0, row)
        mx_ref[pl.ds(r, 1), :] = newrow
        rm = jnp.where(fiota == r, jnp.max(newrow), rm)
        splat(plv, prf, pic, pcf, pbase)
        drow = dp_ref[0, pl.ds(r, 1), :]
        z = jnp.sum(jnp.where(ciota == c, drow, 0.0))
        z = jnp.maximum(z, 0.001)
        radius = jnp.clip(_RADIUS_GAIN / z, _RADIUS_MIN, _RADIUS_MAX)
        sig2 = (0.6 * radius) ** 2
        invc = 1.0 / (2.0 * sig2 + 1e-6)
        base = pl.multiple_of((jnp.clip(r - slab // 2, 0, h - slab) // 8) * 8, 8)
        return (rm, jnp.log(m), r.astype(jnp.float32), invc,
                c.astype(jnp.float32), base)

    init = (rm0, -jnp.inf, 0.0, 0.0, 0.0, 0)
    fin = lax.fori_loop(0, _TOPK, step, init, unroll=False)
    splat(fin[1], fin[2], fin[3], fin[4], fin[5])

    g = jnp.exp(out_ref[0])
    gmn = jnp.min(g)
    gmx = jnp.max(g)
    out_ref[0] = (g - gmn) / (gmx - gmn + 1e-6)


def kernel(voxel_score_map, depth):
    b, ch, h, w = voxel_score_map.shape
    vs = voxel_score_map.reshape(b, h, w)
    dp = depth.reshape(b, h, w)
    out = pl.pallas_call(
        _body,
        grid=(b,),
        in_specs=[
            pl.BlockSpec((1, h, w), lambda i: (i, 0, 0)),
            pl.BlockSpec((1, h, w), lambda i: (i, 0, 0)),
        ],
        out_specs=pl.BlockSpec((1, h, w), lambda i: (i, 0, 0)),
        out_shape=jax.ShapeDtypeStruct((b, h, w), jnp.float32),
        scratch_shapes=[
            pltpu.VMEM((h, w), jnp.float32),
        ],
        compiler_params=pltpu.CompilerParams(
            dimension_semantics=("arbitrary",),
        ),
    )(vs, dp)
    return out.reshape(b, ch, h, w)


# speedup vs baseline: 1.4084x; 1.1095x over previous
import jax
import jax.numpy as jnp
from jax import lax
from jax.experimental import pallas as pl
from jax.experimental.pallas import tpu as pltpu

_TOPK = 80
_RADIUS_GAIN = 14.0
_RADIUS_MIN = 1.5
_RADIUS_MAX = 18.0
_SLAB = 136


def _shift(x, d, axis, fill):
    h, w = x.shape
    if d == 0:
        return x
    if axis == 1:
        pad = jnp.full((h, abs(d)), fill, x.dtype)
        if d > 0:
            return jnp.concatenate([pad, x[:, : w - d]], axis=1)
        return jnp.concatenate([x[:, -d:], pad], axis=1)
    pad = jnp.full((abs(d), w), fill, x.dtype)
    if d > 0:
        return jnp.concatenate([pad, x[: h - d, :]], axis=0)
    return jnp.concatenate([x[-d:, :], pad], axis=0)


def _sum9(x, axis):
    acc = x
    for d in (-4, -3, -2, -1, 1, 2, 3, 4):
        acc = acc + _shift(x, d, axis, 0.0)
    return acc


def _max5(x, axis):
    ninf = -jnp.inf
    m3 = jnp.maximum(x, jnp.maximum(_shift(x, 1, axis, ninf),
                                    _shift(x, -1, axis, ninf)))
    return jnp.maximum(_shift(m3, 1, axis, ninf), _shift(m3, -1, axis, ninf))


def _body(vs_ref, dp_ref, out_ref, mx_ref):
    b, h, w = mx_ref.shape
    nb = h // 8

    ciota = lax.broadcasted_iota(jnp.int32, (1, w), 1)
    xxr = ciota.astype(jnp.float32)
    fiota = (lax.broadcasted_iota(jnp.int32, (8, nb), 0) * nb
             + lax.broadcasted_iota(jnp.int32, (8, nb), 1))
    yyf = lax.broadcasted_iota(jnp.int32, (_SLAB, 1), 0).astype(jnp.float32)

    rm0 = []
    for bi in range(b):
        p = vs_ref[bi]
        occ = (p > 0.2).astype(jnp.float32)
        so = _sum9(_sum9(occ, 1), 0)
        sm = _sum9(_sum9(p, 1), 0)
        prod = so * sm * (1.0 / (81.0 * 81.0))
        mn = jnp.min(prod)
        mxv = jnp.max(prod)
        compact = (prod - mn) / (mxv - mn + 1e-6)
        score = p * compact
        pooled = _max5(_max5(score, 1), 0)
        maxima = jnp.where(score == pooled, score, 0.0)
        mx_ref[bi] = maxima
        rm0.append(jnp.max(maxima.reshape(8, nb, w), axis=2))
        out_ref[bi] = jnp.full((h, w), -jnp.inf, jnp.float32)

    def splat(bi, lv, rfl, ic, cf, base):
        base = pl.multiple_of(base, 8)
        ay = lv - (yyf - rfl) ** 2 * ic
        bx = (xxr - cf) ** 2 * ic
        cur = out_ref[bi, pl.ds(base, _SLAB), :]
        out_ref[bi, pl.ds(base, _SLAB), :] = jnp.maximum(cur, ay - bx)

    def step1(bi, st):
        rm, plv, prfl, pic, pcf, pbase = st
        m = jnp.max(rm)
        r = jnp.min(jnp.where(rm == m, fiota, h))
        row = mx_ref[bi, pl.ds(r, 1), :]
        c = jnp.min(jnp.where(row == m, ciota, w))
        newrow = jnp.where(ciota == c, 0.0, row)
        mx_ref[bi, pl.ds(r, 1), :] = newrow
        rm = jnp.where(fiota == r, jnp.max(newrow), rm)
        splat(bi, plv, prfl, pic, pcf, pbase)
        drow = dp_ref[bi, pl.ds(r, 1), :]
        z = jnp.sum(jnp.where(ciota == c, drow, 0.0))
        z = jnp.maximum(z, 0.001)
        radius = jnp.clip(_RADIUS_GAIN / z, _RADIUS_MIN, _RADIUS_MAX)
        sig2 = (0.6 * radius) ** 2
        invc = 1.0 / (2.0 * sig2 + 1e-6)
        base = pl.multiple_of((jnp.clip(r - _SLAB // 2, 0, h - _SLAB) // 8) * 8,
                              8)
        return (rm, jnp.log(m), (r - base).astype(jnp.float32), invc,
                c.astype(jnp.float32), base)

    def step(i, sts):
        return tuple(step1(bi, sts[bi]) for bi in range(b))

    init = tuple((rm0[bi], -jnp.inf, 0.0, 0.0, 0.0, 0) for bi in range(b))
    fin = lax.fori_loop(0, _TOPK, step, init, unroll=False)

    for bi in range(b):
        splat(bi, *fin[bi][1:])
        g = jnp.exp(out_ref[bi])
        gmn = jnp.min(g)
        gmx = jnp.max(g)
        out_ref[bi] = (g - gmn) / (gmx - gmn + 1e-6)


def kernel(voxel_score_map, depth):
    b, ch, h, w = voxel_score_map.shape
    vs = voxel_score_map.reshape(b, h, w)
    dp = depth.reshape(b, h, w)
    out = pl.pallas_call(
        _body,
        out_shape=jax.ShapeDtypeStruct((b, h, w), jnp.float32),
        scratch_shapes=[
            pltpu.VMEM((b, h, w), jnp.float32),
        ],
    )(vs, dp)
    return out.reshape(b, ch, h, w)
